# Initial kernel scaffold; baseline (speedup 1.0000x reference)
import jax
import jax.numpy as jnp
from jax.experimental import pallas as pl


def kernel(users, items, edge_index, user_table, item_table, W0, b0, W1, b1):
    raise NotImplementedError("write your pallas kernel here")


# speedup vs baseline: 18.0058x; 18.0058x over previous
import functools

import jax
import jax.numpy as jnp
from jax import lax
from jax.experimental import pallas as pl
from jax.experimental.pallas import tpu as pltpu
from jax.experimental.pallas import tpu_sc as plsc

NU = 25000
NN = 50000
D = 64
NE = 800000
NB = 4096
PADH = 600
HALF = NU + PADH
NP = 2 * HALF
E_PAD = 819200
NC, NS = 2, 16

_MESH = plsc.VectorSubcoreMesh(core_axis_name="c", subcore_axis_name="s",
                               num_cores=NC, num_subcores=NS)
_SC_PARAMS = pltpu.CompilerParams(use_tc_tiling_on_sc=False)

H_CH = 2560
H_IT = E_PAD // 32 // H_CH
H_SUB = H_CH // 128


def _deg_body(edge_hbm, zdeg_hbm, deg2_hbm, rraw, hidx, ones_v, deg_sh):
    c = lax.axis_index("c")
    s = lax.axis_index("s")
    w = s * NC + c
    for v in range(8):
        ones_v[pl.ds(v * 16, 16)] = jnp.ones((16,), jnp.float32)
    pltpu.sync_copy(zdeg_hbm, deg_sh.at[pl.ds(s * 3200, 3200)])
    plsc.subcore_barrier()
    iota = lax.iota(jnp.int32, 16)

    def it_body(it, carry):
        base = w * (E_PAD // 32) + it * H_CH
        base2 = pl.multiple_of(jnp.minimum(base, NE - H_CH), 8)
        delta = base - base2
        pltpu.sync_copy(edge_hbm.at[0, pl.ds(base2, H_CH)], rraw)
        for b in range(H_SUB):
            for v in range(8):
                off = b * 128 + v * 16
                offc = pl.multiple_of(
                    jnp.minimum(off + delta, H_CH - 16), 8)
                r = rraw[pl.ds(offc, 16)]
                pos = iota + (it * H_CH + off)
                valid = (base + off + iota) < NE
                bin_ = jnp.where(
                    valid,
                    r + jnp.where(r >= NU, PADH, 0),
                    NU + (pos & 511),
                )
                hidx[b, pl.ds(v * 16, 16)] = bin_
        for b in range(H_SUB):
            pltpu.sync_copy(ones_v, deg_sh.at[hidx.at[b]], add=True)
        return carry

    lax.fori_loop(0, H_IT, it_body, 0)
    plsc.subcore_barrier()
    for r in range(25):
        pltpu.sync_copy(deg_sh.at[pl.ds(s * 3200 + r * 128, 128)],
                        deg2_hbm.at[c, s * 25 + r])


_deg_call = functools.partial(
    pl.kernel,
    out_type=jax.ShapeDtypeStruct((NC, NP // 128, 128), jnp.float32),
    mesh=_MESH,
    compiler_params=_SC_PARAMS,
    scratch_types=[
        pltpu.VMEM((H_CH,), jnp.int32),
        pltpu.VMEM((H_SUB, 128), jnp.int32),
        pltpu.VMEM((128,), jnp.float32),
        pltpu.VMEM_SHARED((NP,), jnp.float32),
    ],
)(_deg_body)

C_CH = 1024
C_IT = E_PAD // 16 // C_CH
C_SUB = C_CH // 128


_GA = 1
_NBUF = 2


def _prop_body(edge_hbm, emb_hbm, zrows_hbm, out_hbm,
               rraw, craw, gidx, sidx, rows, gsem, ssem, psem, acc_sh):
    c = lax.axis_index("c")
    s = lax.axis_index("s")
    lo = c * NU
    pltpu.sync_copy(zrows_hbm, acc_sh.at[pl.ds(s * 1600, 1600)])
    plsc.subcore_barrier()
    iota = lax.iota(jnp.int32, 16)
    ebase = s * (E_PAD // 16)

    def ld_base(it):
        base = ebase + it * C_CH
        return pl.multiple_of(jnp.minimum(base, NE - C_CH), 8)

    def pass_piece(p, b, it):
        base = ebase + it * C_CH
        delta = base - ld_base(it)
        for v in range(8):
            off = b * 128 + v * 16
            offc = pl.multiple_of(jnp.minimum(off + delta, C_CH - 16), 8)
            r = rraw[p, pl.ds(offc, 16)]
            cc = craw[p, pl.ds(offc, 16)]
            pos = iota + (it * C_CH + off)
            valid = (base + off + iota) < NE
            rg = jnp.where(valid, r + jnp.where(r >= NU, PADH, 0),
                           pos & 8191)
            loc = cc - lo
            owned = valid & (loc >= 0) & (loc < NU)
            ce = jnp.where(owned, loc, NU + (cc & 511))
            gidx[p, b, pl.ds(v * 16, 16)] = rg
            sidx[p, b, pl.ds(v * 16, 16)] = ce

    pltpu.sync_copy(edge_hbm.at[0, pl.ds(ld_base(0), C_CH)], rraw.at[0])
    pltpu.sync_copy(edge_hbm.at[1, pl.ds(ld_base(0), C_CH)], craw.at[0])
    for b in range(C_SUB):
        pass_piece(0, b, 0)

    def it_body(it, carry):
        p = lax.rem(it, 2)
        pn = 1 - p
        nxt = ld_base(jnp.minimum(it + 1, C_IT - 1))
        pf_r = pltpu.async_copy(edge_hbm.at[0, pl.ds(nxt, C_CH)],
                                rraw.at[pn], psem)
        pf_c = pltpu.async_copy(edge_hbm.at[1, pl.ds(nxt, C_CH)],
                                craw.at[pn], psem)
        gd = [None] * C_SUB
        sd = [None] * C_SUB
        for b in range(_GA):
            gd[b] = pltpu.async_copy(emb_hbm.at[gidx.at[p, b]], rows.at[b],
                                     gsem)
        for b in range(C_SUB):
            if b == 0:
                pf_r.wait()
                pf_c.wait()
            gd[b].wait()
            sd[b] = pltpu.async_copy(rows.at[b % _NBUF],
                                     acc_sh.at[sidx.at[p, b]], ssem,
                                     add=True)
            nb = b + _GA
            if nb < C_SUB:
                if nb >= _NBUF:
                    sd[nb - _NBUF].wait()
                gd[nb] = pltpu.async_copy(
                    emb_hbm.at[gidx.at[p, nb]],
                    rows.at[nb % _NBUF], gsem)
            pass_piece(pn, b, it + 1)
        for b in range(max(0, C_SUB - _NBUF), C_SUB):
            sd[b].wait()
        return carry

    lax.fori_loop(0, C_IT, it_body, 0)
    plsc.subcore_barrier()
    pltpu.sync_copy(acc_sh.at[pl.ds(s * 1600, 1600)],
                    out_hbm.at[pl.ds(c * HALF + s * 1600, 1600)])


_prop_call = functools.partial(
    pl.kernel,
    out_type=jax.ShapeDtypeStruct((NP, D), jnp.float32),
    mesh=_MESH,
    compiler_params=_SC_PARAMS,
    scratch_types=[
        pltpu.VMEM((2, C_CH), jnp.int32),
        pltpu.VMEM((2, C_CH), jnp.int32),
        pltpu.VMEM((2, C_SUB, 128), jnp.int32),
        pltpu.VMEM((2, C_SUB, 128), jnp.int32),
        pltpu.VMEM((_NBUF, 128, D), jnp.float32),
        pltpu.SemaphoreType.DMA,
        pltpu.SemaphoreType.DMA,
        pltpu.SemaphoreType.DMA,
        pltpu.VMEM_SHARED((HALF, D), jnp.float32),
    ],
)(_prop_body)


def _gather_body(acc_hbm, uidx_hbm, iidx_hbm, gu_hbm, gi_hbm,
                 idx_v, rows_v, sem):
    c = lax.axis_index("c")
    s = lax.axis_index("s")
    w = s * NC + c
    for idx2, gout in ((uidx_hbm, gu_hbm), (iidx_hbm, gi_hbm)):
        pltpu.sync_copy(idx2.at[w], idx_v)
        pltpu.async_copy(acc_hbm.at[idx_v], rows_v, sem).wait()
        pltpu.sync_copy(rows_v, gout.at[pl.ds(w * 128, 128)])


_gather_call = functools.partial(
    pl.kernel,
    out_type=(
        jax.ShapeDtypeStruct((NB, D), jnp.float32),
        jax.ShapeDtypeStruct((NB, D), jnp.float32),
    ),
    mesh=_MESH,
    compiler_params=_SC_PARAMS,
    scratch_types=[
        pltpu.VMEM((128,), jnp.int32),
        pltpu.VMEM((128, D), jnp.float32),
        pltpu.SemaphoreType.DMA,
    ],
)(_gather_body)

NPAIR = NP // 2


def _dinv_body(deg2_ref, dinv_ref):
    deg = deg2_ref[0] + deg2_ref[1]
    dinv_ref[...] = jnp.where(deg > 0, lax.rsqrt(deg), 0.0)


_dinv_call = pl.pallas_call(
    _dinv_body,
    out_shape=jax.ShapeDtypeStruct((NP // 128, 128), jnp.float32),
)

_PRB = 3200


def _scale_body(dinv_ref, emb_ref, out_ref):
    out_ref[...] = dinv_ref[...] * emb_ref[...]


_scale_call = pl.pallas_call(
    _scale_body,
    grid=(NPAIR // _PRB,),
    in_specs=[pl.BlockSpec((_PRB, 128), lambda i: (i, 0)),
              pl.BlockSpec((_PRB, 128), lambda i: (i, 0))],
    out_specs=pl.BlockSpec((_PRB, 128), lambda i: (i, 0)),
    out_shape=jax.ShapeDtypeStruct((NPAIR, 128), jnp.float32),
)


def _mid_body(acc_ref, dinv_ref, w_ref, b_ref, out_ref):
    dv = dinv_ref[...]
    x = dv * acc_ref[...]
    h = jnp.dot(x, w_ref[...], preferred_element_type=jnp.float32)
    h = jnp.maximum(h + b_ref[...], 0.0)
    out_ref[...] = dv * h


_mid_call = pl.pallas_call(
    _mid_body,
    grid=(NPAIR // _PRB,),
    in_specs=[pl.BlockSpec((_PRB, 128), lambda i: (i, 0)),
              pl.BlockSpec((_PRB, 128), lambda i: (i, 0)),
              pl.BlockSpec((128, 128), lambda i: (0, 0)),
              pl.BlockSpec((1, 128), lambda i: (0, 0))],
    out_specs=pl.BlockSpec((_PRB, 128), lambda i: (i, 0)),
    out_shape=jax.ShapeDtypeStruct((NPAIR, 128), jnp.float32),
)


def _final_body(gu_ref, gi_ref, w_ref, b_ref, sel_ref, out_ref):
    u = jnp.dot(gu_ref[...], w_ref[...],
                preferred_element_type=jnp.float32) + b_ref[...]
    i = jnp.dot(gi_ref[...], w_ref[...],
                preferred_element_type=jnp.float32) + b_ref[...]
    out_ref[...] = jnp.dot(u * i, sel_ref[...],
                           preferred_element_type=jnp.float32)


_final_call = pl.pallas_call(
    _final_body,
    out_shape=jax.ShapeDtypeStruct((NB // 2, 2), jnp.float32),
)


def kernel(users, items, edge_index, user_table, item_table, W0, b0, W1, b1):
    f32 = jnp.float32
    edge32 = edge_index.astype(jnp.int32)
    zh = jnp.zeros((PADH, D), f32)
    emb_p = jnp.concatenate([user_table, zh, item_table, zh], axis=0)
    zrows = jnp.zeros((1600, D), f32)
    zdeg = jnp.zeros((3200,), f32)
    uidx = users.astype(jnp.int32).reshape(NC * NS, 128)
    iidx = (items.astype(jnp.int32) + HALF).reshape(NC * NS, 128)
    eye2 = jnp.eye(2, dtype=f32)
    W0T2 = jnp.kron(eye2, W0.T)
    W1T2 = jnp.kron(eye2, W1.T)
    b0_2 = jnp.tile(b0, 2).reshape(1, 128)
    b1_2 = jnp.tile(b1, 2).reshape(1, 128)
    sel = jnp.kron(eye2, jnp.ones((D, 1), f32))

    deg2 = _deg_call(edge32, zdeg)
    dinv_nodes = _dinv_call(deg2)
    dinv128 = jnp.broadcast_to(dinv_nodes.reshape(NP, 1),
                               (NP, D)).reshape(NPAIR, 128)
    emb0p = _scale_call(dinv128, emb_p.reshape(NPAIR, 128))
    acc0 = _prop_call(edge32, emb0p.reshape(NP, D), zrows)
    h1 = _mid_call(acc0.reshape(NPAIR, 128), dinv128, W0T2, b0_2)
    acc1 = _prop_call(edge32, h1.reshape(NP, D), zrows)
    fin = _scale_call(dinv128, acc1.reshape(NPAIR, 128))
    gu, gi = _gather_call(fin.reshape(NP, D), uidx, iidx)
    sc = _final_call(gu.reshape(NB // 2, 128), gi.reshape(NB // 2, 128),
                     W1T2, b1_2, sel)
    return sc.reshape(NB)
